# baseline (speedup 1.0000x reference)
import functools

import jax
import jax.numpy as jnp
from jax import lax
from jax.experimental import pallas as pl
from jax.experimental.pallas import tpu as pltpu
from jax.experimental.pallas import tpu_sc as plsc

D = 1024
F = 4096
NE = 8
TOPK = 2
RT = 128
FB = 256
NF = F // FB


def _router_body(x_ref, g_ref, idx_ref, wts_ref):
    x = x_ref[...]
    g = g_ref[...]
    logits = jnp.dot(x, g, preferred_element_type=jnp.float32)
    t = logits.shape[0]
    cols = jax.lax.broadcasted_iota(jnp.int32, (t, NE), 1)
    m1 = jnp.max(logits, axis=1)
    i1 = jnp.min(jnp.where(logits == m1[:, None], cols, NE), axis=1)
    masked = jnp.where(cols == i1[:, None], -jnp.inf, logits)
    m2 = jnp.max(masked, axis=1)
    i2 = jnp.min(jnp.where(masked == m2[:, None], cols, NE), axis=1)
    e2 = jnp.exp(m2 - m1)
    w1 = 1.0 / (1.0 + e2)
    w2 = e2 / (1.0 + e2)
    idx_ref[...] = jnp.stack([i1, i2])
    wts_ref[...] = jnp.stack([w1, w2])


def _router(xt, gate_kernel):
    t = xt.shape[0]
    return pl.pallas_call(
        _router_body,
        out_shape=(jax.ShapeDtypeStruct((TOPK, t), jnp.int32),
                   jax.ShapeDtypeStruct((TOPK, t), jnp.float32)),
    )(xt, gate_kernel)


def _gemm_body(lo_ref, cnt_ref, xs_ref, wi0_ref, wi1_ref, wo_ref, out_ref):
    s = pl.program_id(0)
    e = s // NF
    f = s % NF
    w0 = wi0_ref[0].astype(jnp.bfloat16)
    w1 = wi1_ref[0].astype(jnp.bfloat16)
    wov = wo_ref[0].astype(jnp.bfloat16)
    lo = lo_ref[e]
    ntile = cnt_ref[e] // RT

    def tile_body(j, carry):
        row = pl.multiple_of(lo + j * RT, RT)
        x = xs_ref[pl.ds(row, RT), :].astype(jnp.bfloat16)
        h0 = jnp.dot(x, w0, preferred_element_type=jnp.float32)
        h1 = jnp.dot(x, w1, preferred_element_type=jnp.float32)
        g = (jax.nn.gelu(h0) * h1).astype(jnp.bfloat16)
        contrib = jnp.dot(g, wov, preferred_element_type=jnp.float32)

        @pl.when(f == 0)
        def _():
            out_ref[pl.ds(row, RT), :] = contrib

        @pl.when(f != 0)
        def _():
            out_ref[pl.ds(row, RT), :] += contrib

        return carry

    jax.lax.fori_loop(0, ntile, tile_body, 0)


def _grouped_gemm(xs, wi_0, wi_1, wo, group_lo, group_cnt, nt):
    ntot = nt * RT
    grid_spec = pltpu.PrefetchScalarGridSpec(
        num_scalar_prefetch=2,
        grid=(NE * NF,),
        in_specs=[
            pl.BlockSpec((ntot, D), lambda s, lo, cnt: (0, 0)),
            pl.BlockSpec((1, D, FB), lambda s, lo, cnt: (s // NF, 0, s % NF)),
            pl.BlockSpec((1, D, FB), lambda s, lo, cnt: (s // NF, 0, s % NF)),
            pl.BlockSpec((1, FB, D), lambda s, lo, cnt: (s // NF, s % NF, 0)),
        ],
        out_specs=pl.BlockSpec((ntot, D), lambda s, lo, cnt: (0, 0)),
    )
    return pl.pallas_call(
        _gemm_body,
        grid_spec=grid_spec,
        out_shape=jax.ShapeDtypeStruct((ntot, D), jnp.float32),
        compiler_params=pltpu.CompilerParams(
            dimension_semantics=("arbitrary",),
            vmem_limit_bytes=100 * 1024 * 1024,
        ),
    )(group_lo, group_cnt, xs, wi_0, wi_1, wo)


def _dyn_gather(v, idx):
    dn = lax.GatherDimensionNumbers(
        offset_dims=(), collapsed_slice_dims=(0,), start_index_map=(0,))
    return lax.gather(v, idx[:, None], dn, (1,),
                      mode=lax.GatherScatterMode.PROMISE_IN_BOUNDS)


def _cumsum16(x):
    lane = lax.iota(jnp.int32, 16)
    y = x
    for k in (1, 2, 4, 8):
        sh = jnp.where(lane >= k, _dyn_gather(y, jnp.maximum(lane - k, 0)), 0)
        y = y + sh
    return y


_T = 2048
_NDISP = _T * TOPK
_NT = (_NDISP + NE * (RT - 1) + RT - 1) // RT
_NTOT = _NT * RT


def _make_binning():
    mesh = plsc.VectorSubcoreMesh(core_axis_name="c", subcore_axis_name="s")

    @functools.partial(
        pl.kernel,
        mesh=mesh,
        out_type=(
            jax.ShapeDtypeStruct((_NDISP,), jnp.int32),
            jax.ShapeDtypeStruct((_NTOT, D), jnp.float32),
            jax.ShapeDtypeStruct((16,), jnp.int32),
            jax.ShapeDtypeStruct((16,), jnp.int32),
        ),
        scratch_types=[
            pltpu.VMEM((256,), jnp.int32),
            pltpu.VMEM((32,), jnp.int32),
            pltpu.VMEM_SHARED((512,), jnp.int32),
            pltpu.VMEM((512,), jnp.int32),
            pltpu.VMEM((128,), jnp.int32),
            pltpu.VMEM((16, D), jnp.float32),
            pltpu.VMEM((16,), jnp.int32),
            pltpu.VMEM((16,), jnp.int32),
            pltpu.SemaphoreType.DMA,
        ],
    )
    def binning(eidx, x, pos_hbm, xs_hbm, glo_hbm, gcnt_hbm,
                idx_v, stage, shared, all_v, posflat, rows_v,
                gstage, gstage2, sem):
        c = lax.axis_index("c")
        s = lax.axis_index("s")
        w0 = 2 * s + c
        lane = lax.iota(jnp.int32, 16)

        pltpu.sync_copy(eidx.at[pl.ds(256 * s, 256)], idx_v)
        cnts = [jnp.zeros((16,), jnp.int32), jnp.zeros((16,), jnp.int32)]
        for j in range(16):
            ids = idx_v[pl.ds(j * 16, 16)]
            h = jnp.zeros((16,), jnp.int32)
            for e in range(NE):
                pc = _cumsum16(jnp.where(ids == e, 1, 0))
                tot = _dyn_gather(pc, lane * 0 + 15)
                h = h + jnp.where(lane == e, tot, 0)
            cnts[j // 8] = cnts[j // 8] + h
        stage[pl.ds(0, 16)] = cnts[0]
        stage[pl.ds(16, 16)] = cnts[1]
        pltpu.sync_copy(stage, shared.at[pl.ds(32 * s, 32)])
        plsc.subcore_barrier()

        pltpu.sync_copy(shared, all_v)

        def _sum_rows(i, acc):
            return acc + all_v[pl.ds(i * 16, 16)]

        total = lax.fori_loop(0, 32, _sum_rows, jnp.zeros((16,), jnp.int32))
        padded = ((total + (RT - 1)) >> 7) << 7
        group_start = _cumsum16(padded) - padded
        prior = lax.fori_loop(0, w0, _sum_rows, jnp.zeros((16,), jnp.int32))
        run = group_start + prior

        @pl.when(w0 == 0)
        def _():
            gstage[...] = group_start
            gstage2[...] = padded
            pltpu.sync_copy(gstage, glo_hbm)
            pltpu.sync_copy(gstage2, gcnt_hbm)

        dests = []
        for j in range(8):
            ids = idx_v[pl.ds(128 * c + j * 16, 16)]
            dest = jnp.zeros((16,), jnp.int32)
            newrun = run
            for e in range(NE):
                m = ids == e
                pc = _cumsum16(jnp.where(m, 1, 0))
                re_vec = _dyn_gather(run, lane * 0 + e)
                dest = jnp.where(m, re_vec + pc - 1, dest)
                tot = _dyn_gather(pc, lane * 0 + 15)
                newrun = newrun + jnp.where(lane == e, tot, 0)
            posflat[pl.ds(j * 16, 16)] = dest
            dests.append(dest)
            run = newrun
        pltpu.sync_copy(posflat, pos_hbm.at[pl.ds(128 * w0, 128)])

        tb = (w0 % 16) * 128
        for j in range(8):
            pltpu.sync_copy(x.at[pl.ds(tb + 16 * j, 16)], rows_v)
            pltpu.async_copy(rows_v, xs_hbm.at[dests[j]], sem).wait()

    return binning


def _make_combine():
    mesh = plsc.VectorSubcoreMesh(core_axis_name="c", subcore_axis_name="s")

    @functools.partial(
        pl.kernel,
        mesh=mesh,
        out_type=jax.ShapeDtypeStruct((_T, D), jnp.float32),
        scratch_types=[
            pltpu.VMEM((64,), jnp.int32),
            pltpu.VMEM((64,), jnp.int32),
            pltpu.VMEM((64,), jnp.float32),
            pltpu.VMEM((64,), jnp.float32),
            pltpu.VMEM((2, 16, D), jnp.float32),
            pltpu.VMEM((2, 16, D), jnp.float32),
            pltpu.VMEM((16, D), jnp.float32),
            pltpu.SemaphoreType.DMA,
            pltpu.SemaphoreType.DMA,
            pltpu.SemaphoreType.DMA,
        ],
    )
    def combine(pos, wts, eo, out_hbm,
                p0, p1, w0r, w1r, r0, r1, ob, sem0, sem1, semo):
        c = lax.axis_index("c")
        s = lax.axis_index("s")
        tb = 64 * (2 * s + c)
        lane = lax.iota(jnp.int32, 16)
        pltpu.sync_copy(pos.at[pl.ds(tb, 64)], p0)
        pltpu.sync_copy(pos.at[pl.ds(_T + tb, 64)], p1)
        pltpu.sync_copy(wts.at[pl.ds(tb, 64)], w0r)
        pltpu.sync_copy(wts.at[pl.ds(_T + tb, 64)], w1r)

        def _start(cc, buf):
            i0 = p0[pl.ds(cc * 16, 16)]
            i1 = p1[pl.ds(cc * 16, 16)]
            cp0 = pltpu.async_copy(eo.at[i0], r0.at[buf], sem0)
            cp1 = pltpu.async_copy(eo.at[i1], r1.at[buf], sem1)
            return cp0, cp1

        pend = _start(0, 0)
        for cc in range(4):
            buf = cc % 2
            pend[0].wait()
            pend[1].wait()
            if cc < 3:
                pend = _start(cc + 1, 1 - buf)
            wv0 = w0r[pl.ds(cc * 16, 16)]
            wv1 = w1r[pl.ds(cc * 16, 16)]

            def tok_body(j, carry):
                jsplat = lane * 0 + j
                a0 = _dyn_gather(wv0, jsplat)
                a1 = _dyn_gather(wv1, jsplat)
                for dv in range(64):
                    sl = pl.ds(dv * 16, 16)
                    ob[j, sl] = (a0 * r0[buf, j, sl]
                                 + a1 * r1[buf, j, sl])
                return carry

            lax.fori_loop(0, 16, tok_body, 0)
            cpo = pltpu.async_copy(ob, out_hbm.at[pl.ds(tb + cc * 16, 16)],
                                   semo)
            cpo.wait()

    return combine


_BINNING = _make_binning()
_COMBINE = _make_combine()


def kernel(inputs, decoder_segment_ids, decoder_positions, gate_kernel,
           wi_0, wi_1, wo):
    b, s, d = inputs.shape
    xt = inputs.reshape(b * s, d)

    idx_kt, wts_kt = _router(xt, gate_kernel)

    pos, xs, group_lo, group_cnt = _BINNING(idx_kt.reshape(-1), xt)

    eo = _grouped_gemm(xs, wi_0, wi_1, wo, group_lo, group_cnt, _NT)

    out = _COMBINE(pos, wts_kt.reshape(-1), eo)
    return out.reshape(b, s, d)


# speedup vs baseline: 1.3482x; 1.3482x over previous
import functools

import jax
import jax.numpy as jnp
from jax import lax
from jax.experimental import pallas as pl
from jax.experimental.pallas import tpu as pltpu
from jax.experimental.pallas import tpu_sc as plsc

D = 1024
F = 4096
NE = 8
TOPK = 2
RT = 128
FB = 512
NF = F // FB


def _router_body(x_ref, g_ref, idx_ref, wts_ref):
    x = x_ref[...]
    g = g_ref[...]
    logits = jnp.dot(x, g, preferred_element_type=jnp.float32)
    t = logits.shape[0]
    cols = jax.lax.broadcasted_iota(jnp.int32, (t, NE), 1)
    m1 = jnp.max(logits, axis=1)
    i1 = jnp.min(jnp.where(logits == m1[:, None], cols, NE), axis=1)
    masked = jnp.where(cols == i1[:, None], -jnp.inf, logits)
    m2 = jnp.max(masked, axis=1)
    i2 = jnp.min(jnp.where(masked == m2[:, None], cols, NE), axis=1)
    e2 = jnp.exp(m2 - m1)
    w1 = 1.0 / (1.0 + e2)
    w2 = e2 / (1.0 + e2)
    idx_ref[...] = jnp.stack([i1, i2])
    wts_ref[...] = jnp.stack([w1, w2])


def _router(xt, gate_kernel):
    t = xt.shape[0]
    return pl.pallas_call(
        _router_body,
        out_shape=(jax.ShapeDtypeStruct((TOPK, t), jnp.int32),
                   jax.ShapeDtypeStruct((TOPK, t), jnp.float32)),
    )(xt, gate_kernel)


def _gemm_body(lo_ref, cnt_ref, xs_ref, wi0_ref, wi1_ref, wo_ref, out_ref):
    s = pl.program_id(0)
    e = s // NF
    f = s % NF
    w0 = wi0_ref[0].astype(jnp.bfloat16)
    w1 = wi1_ref[0].astype(jnp.bfloat16)
    wov = wo_ref[0].astype(jnp.bfloat16)
    lo = lo_ref[e]
    ntile = cnt_ref[e] // RT

    def tile_body(j, carry):
        row = pl.multiple_of(lo + j * RT, RT)
        x = xs_ref[pl.ds(row, RT), :].astype(jnp.bfloat16)
        h0 = jnp.dot(x, w0, preferred_element_type=jnp.float32)
        h1 = jnp.dot(x, w1, preferred_element_type=jnp.float32)
        g = (jax.nn.gelu(h0) * h1).astype(jnp.bfloat16)
        contrib = jnp.dot(g, wov, preferred_element_type=jnp.float32)

        @pl.when(f == 0)
        def _():
            out_ref[pl.ds(row, RT), :] = contrib

        @pl.when(f != 0)
        def _():
            out_ref[pl.ds(row, RT), :] += contrib

        return carry

    jax.lax.fori_loop(0, ntile, tile_body, 0)


def _grouped_gemm(xs, wi_0, wi_1, wo, group_lo, group_cnt, nt):
    ntot = nt * RT
    grid_spec = pltpu.PrefetchScalarGridSpec(
        num_scalar_prefetch=2,
        grid=(NE * NF,),
        in_specs=[
            pl.BlockSpec((ntot, D), lambda s, lo, cnt: (0, 0)),
            pl.BlockSpec((1, D, FB), lambda s, lo, cnt: (s // NF, 0, s % NF)),
            pl.BlockSpec((1, D, FB), lambda s, lo, cnt: (s // NF, 0, s % NF)),
            pl.BlockSpec((1, FB, D), lambda s, lo, cnt: (s // NF, s % NF, 0)),
        ],
        out_specs=pl.BlockSpec((ntot, D), lambda s, lo, cnt: (0, 0)),
    )
    return pl.pallas_call(
        _gemm_body,
        grid_spec=grid_spec,
        out_shape=jax.ShapeDtypeStruct((ntot, D), jnp.float32),
        compiler_params=pltpu.CompilerParams(
            dimension_semantics=("arbitrary",),
            vmem_limit_bytes=100 * 1024 * 1024,
        ),
    )(group_lo, group_cnt, xs, wi_0, wi_1, wo)


def _dyn_gather(v, idx):
    dn = lax.GatherDimensionNumbers(
        offset_dims=(), collapsed_slice_dims=(0,), start_index_map=(0,))
    return lax.gather(v, idx[:, None], dn, (1,),
                      mode=lax.GatherScatterMode.PROMISE_IN_BOUNDS)


def _cumsum16(x):
    lane = lax.iota(jnp.int32, 16)
    y = x
    for k in (1, 2, 4, 8):
        sh = jnp.where(lane >= k, _dyn_gather(y, jnp.maximum(lane - k, 0)), 0)
        y = y + sh
    return y


_T = 2048
_NDISP = _T * TOPK
_NT = (_NDISP + NE * (RT - 1) + RT - 1) // RT
_NTOT = _NT * RT


def _make_binning():
    mesh = plsc.VectorSubcoreMesh(core_axis_name="c", subcore_axis_name="s")

    @functools.partial(
        pl.kernel,
        mesh=mesh,
        out_type=(
            jax.ShapeDtypeStruct((_NDISP,), jnp.int32),
            jax.ShapeDtypeStruct((_NTOT, D), jnp.float32),
            jax.ShapeDtypeStruct((16,), jnp.int32),
            jax.ShapeDtypeStruct((16,), jnp.int32),
        ),
        scratch_types=[
            pltpu.VMEM((256,), jnp.int32),
            pltpu.VMEM((32,), jnp.int32),
            pltpu.VMEM_SHARED((512,), jnp.int32),
            pltpu.VMEM((512,), jnp.int32),
            pltpu.VMEM((128,), jnp.int32),
            pltpu.VMEM((2, 16, D), jnp.float32),
            pltpu.VMEM((16,), jnp.int32),
            pltpu.VMEM((16,), jnp.int32),
            pltpu.SemaphoreType.DMA,
            pltpu.SemaphoreType.DMA,
        ],
    )
    def binning(eidx, x, pos_hbm, xs_hbm, glo_hbm, gcnt_hbm,
                idx_v, stage, shared, all_v, posflat, rows_v,
                gstage, gstage2, sem, semi):
        c = lax.axis_index("c")
        s = lax.axis_index("s")
        w0 = 2 * s + c
        tb = (w0 % 16) * 128
        lane = lax.iota(jnp.int32, 16)
        cin = pltpu.async_copy(x.at[pl.ds(tb, 16)], rows_v.at[0], semi)

        pltpu.sync_copy(eidx.at[pl.ds(256 * s, 256)], idx_v)
        cnts = [jnp.zeros((16,), jnp.int32), jnp.zeros((16,), jnp.int32)]
        for j in range(16):
            ids = idx_v[pl.ds(j * 16, 16)]
            h = jnp.zeros((16,), jnp.int32)
            for e in range(NE):
                pc = _cumsum16(jnp.where(ids == e, 1, 0))
                tot = _dyn_gather(pc, lane * 0 + 15)
                h = h + jnp.where(lane == e, tot, 0)
            cnts[j // 8] = cnts[j // 8] + h
        stage[pl.ds(0, 16)] = cnts[0]
        stage[pl.ds(16, 16)] = cnts[1]
        pltpu.sync_copy(stage, shared.at[pl.ds(32 * s, 32)])
        plsc.subcore_barrier()

        pltpu.sync_copy(shared, all_v)

        def _sum_rows(i, acc):
            return acc + all_v[pl.ds(i * 16, 16)]

        total = lax.fori_loop(0, 32, _sum_rows, jnp.zeros((16,), jnp.int32))
        padded = ((total + (RT - 1)) >> 7) << 7
        group_start = _cumsum16(padded) - padded
        prior = lax.fori_loop(0, w0, _sum_rows, jnp.zeros((16,), jnp.int32))
        run = group_start + prior

        @pl.when(w0 == 0)
        def _():
            gstage[...] = group_start
            gstage2[...] = padded
            pltpu.sync_copy(gstage, glo_hbm)
            pltpu.sync_copy(gstage2, gcnt_hbm)

        dests = []
        for j in range(8):
            ids = idx_v[pl.ds(128 * c + j * 16, 16)]
            dest = jnp.zeros((16,), jnp.int32)
            newrun = run
            for e in range(NE):
                m = ids == e
                pc = _cumsum16(jnp.where(m, 1, 0))
                re_vec = _dyn_gather(run, lane * 0 + e)
                dest = jnp.where(m, re_vec + pc - 1, dest)
                tot = _dyn_gather(pc, lane * 0 + 15)
                newrun = newrun + jnp.where(lane == e, tot, 0)
            posflat[pl.ds(j * 16, 16)] = dest
            dests.append(dest)
            run = newrun
        pltpu.sync_copy(posflat, pos_hbm.at[pl.ds(128 * w0, 128)])

        for j in range(8):
            buf = j % 2
            cin.wait()
            if j < 7:
                cin = pltpu.async_copy(x.at[pl.ds(tb + 16 * (j + 1), 16)],
                                       rows_v.at[1 - buf], semi)
            pltpu.async_copy(rows_v.at[buf], xs_hbm.at[dests[j]], sem).wait()

    return binning


def _make_combine():
    mesh = plsc.VectorSubcoreMesh(core_axis_name="c", subcore_axis_name="s")

    @functools.partial(
        pl.kernel,
        mesh=mesh,
        out_type=jax.ShapeDtypeStruct((_T, D), jnp.float32),
        scratch_types=[
            pltpu.VMEM((64,), jnp.int32),
            pltpu.VMEM((64,), jnp.int32),
            pltpu.VMEM((64,), jnp.float32),
            pltpu.VMEM((64,), jnp.float32),
            pltpu.VMEM((2, 16, D), jnp.float32),
            pltpu.VMEM((2, 16, D), jnp.float32),
            pltpu.VMEM((16, D), jnp.float32),
            pltpu.SemaphoreType.DMA,
            pltpu.SemaphoreType.DMA,
            pltpu.SemaphoreType.DMA,
        ],
    )
    def combine(pos, wts, eo, out_hbm,
                p0, p1, w0r, w1r, r0, r1, ob, sem0, sem1, semo):
        c = lax.axis_index("c")
        s = lax.axis_index("s")
        tb = 64 * (2 * s + c)
        lane = lax.iota(jnp.int32, 16)
        pltpu.sync_copy(pos.at[pl.ds(tb, 64)], p0)
        pltpu.sync_copy(pos.at[pl.ds(_T + tb, 64)], p1)
        pltpu.sync_copy(wts.at[pl.ds(tb, 64)], w0r)
        pltpu.sync_copy(wts.at[pl.ds(_T + tb, 64)], w1r)

        def _start(cc, buf):
            i0 = p0[pl.ds(cc * 16, 16)]
            i1 = p1[pl.ds(cc * 16, 16)]
            cp0 = pltpu.async_copy(eo.at[i0], r0.at[buf], sem0)
            cp1 = pltpu.async_copy(eo.at[i1], r1.at[buf], sem1)
            return cp0, cp1

        pend = _start(0, 0)
        for cc in range(4):
            buf = cc % 2
            pend[0].wait()
            pend[1].wait()
            if cc < 3:
                pend = _start(cc + 1, 1 - buf)
            wv0 = w0r[pl.ds(cc * 16, 16)]
            wv1 = w1r[pl.ds(cc * 16, 16)]

            def tok_body(j, carry):
                jsplat = lane * 0 + j
                a0 = _dyn_gather(wv0, jsplat)
                a1 = _dyn_gather(wv1, jsplat)
                for dv in range(64):
                    sl = pl.ds(dv * 16, 16)
                    ob[j, sl] = (a0 * r0[buf, j, sl]
                                 + a1 * r1[buf, j, sl])
                return carry

            lax.fori_loop(0, 16, tok_body, 0)
            cpo = pltpu.async_copy(ob, out_hbm.at[pl.ds(tb + cc * 16, 16)],
                                   semo)
            cpo.wait()

    return combine


_BINNING = _make_binning()
_COMBINE = _make_combine()


def kernel(inputs, decoder_segment_ids, decoder_positions, gate_kernel,
           wi_0, wi_1, wo):
    b, s, d = inputs.shape
    xt = inputs.reshape(b * s, d)

    idx_kt, wts_kt = _router(xt, gate_kernel)

    pos, xs, group_lo, group_cnt = _BINNING(idx_kt.reshape(-1), xt)

    eo = _grouped_gemm(xs, wi_0, wi_1, wo, group_lo, group_cnt, _NT)

    out = _COMBINE(pos, wts_kt.reshape(-1), eo)
    return out.reshape(b, s, d)
